# Initial kernel scaffold; baseline (speedup 1.0000x reference)
import jax
import jax.numpy as jnp
from jax.experimental import pallas as pl


def kernel(hn, he, edge_index, fe, fes, norm, ev_W1, ev_b1, ev_W2, ev_b2, fc_W1, fc_W2, nu_W1, nu_b1, nu_W2, nu_b2):
    raise NotImplementedError("write your pallas kernel here")


# speedup vs baseline: 1.6238x; 1.6238x over previous
import functools

import jax
import jax.numpy as jnp
from jax import lax
from jax.experimental import pallas as pl
from jax.experimental.pallas import tpu as pltpu
from jax.experimental.pallas import tpu_sc as plsc

N_NODES = 10000
E = 160000
D = 128
D_VAL = 16
NUM_FES = 16
H1 = 512
H_FC = 64

NC = 2
NS = 16
NW = NC * NS

CHUNK = 128
N_CHUNKS = E // CHUNK
GATHER_ITERS = -(-N_CHUNKS // NW)
SC_CHUNKS = (E // 2) // CHUNK
SCATTER_ITERS = -(-SC_CHUNKS // NS)
ZROWS = 624
ZTAIL_OFF = ZROWS * NS
ZTAIL = N_NODES - ZTAIL_OFF


def _gather_body(hn_hbm, src_hbm, dst_hbm, hs_hbm, hd_hbm,
                 idx_a, rows_a, idx_b, rows_b, sem_a, sem_b):
    wid = lax.axis_index("s") * NC + lax.axis_index("c")

    def step(t, _):
        chunk = t * NW + wid

        @pl.when(chunk < N_CHUNKS)
        def _():
            base = chunk * CHUNK
            pltpu.sync_copy(src_hbm.at[pl.ds(base, CHUNK)], idx_a)
            pltpu.sync_copy(dst_hbm.at[pl.ds(base, CHUNK)], idx_b)
            cp_a = pltpu.async_copy(hn_hbm.at[idx_a], rows_a, sem_a)
            cp_b = pltpu.async_copy(hn_hbm.at[idx_b], rows_b, sem_b)
            cp_a.wait()
            pltpu.sync_copy(rows_a, hs_hbm.at[pl.ds(base, CHUNK)])
            cp_b.wait()
            pltpu.sync_copy(rows_b, hd_hbm.at[pl.ds(base, CHUNK)])
        return None

    lax.fori_loop(0, GATHER_ITERS, step, None)


@jax.jit
def _sc_gather(hn, src, dst):
    mesh = plsc.VectorSubcoreMesh(core_axis_name="c", subcore_axis_name="s")
    return pl.kernel(
        _gather_body,
        out_type=(
            jax.ShapeDtypeStruct((E, D), jnp.float32),
            jax.ShapeDtypeStruct((E, D), jnp.float32),
        ),
        mesh=mesh,
        scratch_types=[
            pltpu.VMEM((CHUNK,), jnp.int32),
            pltpu.VMEM((CHUNK, D), jnp.float32),
            pltpu.VMEM((CHUNK,), jnp.int32),
            pltpu.VMEM((CHUNK, D), jnp.float32),
            pltpu.SemaphoreType.DMA,
            pltpu.SemaphoreType.DMA,
        ],
    )(hn, src, dst)


def _scatter_body(hen_s_hbm, dst_hbm, zero_hbm, out_hbm,
                  idx_v, rows_v, acc):
    cid = lax.axis_index("c")
    sid = lax.axis_index("s")

    r0 = sid * ZROWS
    pltpu.sync_copy(zero_hbm.at[pl.ds(r0, ZROWS)], acc.at[pl.ds(r0, ZROWS)])

    @pl.when(sid == 0)
    def _():
        pltpu.sync_copy(zero_hbm.at[pl.ds(ZTAIL_OFF, ZTAIL)],
                        acc.at[pl.ds(ZTAIL_OFF, ZTAIL)])
    plsc.subcore_barrier()

    def step(t, _):
        chunk = t * NS + sid

        @pl.when(chunk < SC_CHUNKS)
        def _():
            base = cid * (E // 2) + chunk * CHUNK
            pltpu.sync_copy(dst_hbm.at[pl.ds(base, CHUNK)], idx_v)
            pltpu.sync_copy(hen_s_hbm.at[pl.ds(base, CHUNK)], rows_v)
            pltpu.sync_copy(rows_v, acc.at[idx_v], add=True)
        return None

    lax.fori_loop(0, SCATTER_ITERS, step, None)
    plsc.subcore_barrier()
    pltpu.sync_copy(acc.at[pl.ds(r0, ZROWS)], out_hbm.at[cid, pl.ds(r0, ZROWS)])

    @pl.when(sid == 0)
    def _():
        pltpu.sync_copy(acc.at[pl.ds(ZTAIL_OFF, ZTAIL)],
                        out_hbm.at[cid, pl.ds(ZTAIL_OFF, ZTAIL)])


@jax.jit
def _sc_scatter(hen_s, dst):
    mesh = plsc.VectorSubcoreMesh(core_axis_name="c", subcore_axis_name="s")
    zero = jnp.zeros((N_NODES, D), jnp.float32)
    return pl.kernel(
        _scatter_body,
        out_type=jax.ShapeDtypeStruct((NC, N_NODES, D), jnp.float32),
        mesh=mesh,
        scratch_types=[
            pltpu.VMEM((CHUNK,), jnp.int32),
            pltpu.VMEM((CHUNK, D), jnp.float32),
            pltpu.VMEM_SHARED((N_NODES, D), jnp.float32),
        ],
    )(hen_s, dst, zero)


BE = 800


def _edge_body(he, hs, hd, fe, fes, norm,
               w1a, w1b, w1c, b1, w2, b2, fcw1, fcw2,
               hen_out, hen_s_out):
    h1 = jnp.dot(he[...], w1a[...], preferred_element_type=jnp.float32)
    h1 += jnp.dot(hs[...], w1b[...], preferred_element_type=jnp.float32)
    h1 += jnp.dot(hd[...], w1c[...], preferred_element_type=jnp.float32)
    h1 = jnp.maximum(h1 + b1[...], 0.0)
    v = jnp.dot(h1, w2[...], preferred_element_type=jnp.float32) + b2[...]
    h = jnp.maximum(jnp.dot(fes[...], fcw1[...],
                            preferred_element_type=jnp.float32) * 0.25, 0.0)
    g = jnp.dot(h, fcw2[...], preferred_element_type=jnp.float32)
    acc = v[:, 0:1] * g[:, 0:D]
    for j in range(1, D_VAL):
        acc += v[:, j:j + 1] * g[:, j * D:(j + 1) * D]
    heu = fe[...] * acc * (1.0 / 32.0)
    hen = he[...] + heu
    hen_out[...] = hen
    hen_s_out[...] = hen * norm[...]


@jax.jit
def _tc_edge(he, hs, hd, fe, fes, norm, ev_W1, ev_b1, ev_W2, ev_b2,
             fc_W1, fc_W2):
    grid = (E // BE,)
    eb = lambda w: pl.BlockSpec((BE, w), lambda i: (i, 0))
    full = lambda a, b: pl.BlockSpec((a, b), lambda i: (0, 0))
    return pl.pallas_call(
        _edge_body,
        grid=grid,
        in_specs=[
            eb(D), eb(D), eb(D), eb(1), eb(NUM_FES), eb(1),
            full(D, H1), full(D, H1), full(D, H1), full(1, H1),
            full(H1, D_VAL), full(1, D_VAL),
            full(NUM_FES, H_FC), full(H_FC, D_VAL * D),
        ],
        out_specs=[eb(D), eb(D)],
        out_shape=(
            jax.ShapeDtypeStruct((E, D), jnp.float32),
            jax.ShapeDtypeStruct((E, D), jnp.float32),
        ),
    )(he, hs, hd, fe, fes, norm.reshape(E, 1),
      ev_W1[:D], ev_W1[D:2 * D], ev_W1[2 * D:], ev_b1.reshape(1, H1),
      ev_W2, ev_b2.reshape(1, D_VAL), fc_W1, fc_W2)


BN = 2000


def _node_body(hn, p0, p1, w1a, w1b, b1, w2, b2, out):
    ntmp = p0[0] + p1[0]
    h1 = jnp.dot(hn[...], w1a[...], preferred_element_type=jnp.float32)
    h1 += jnp.dot(ntmp, w1b[...], preferred_element_type=jnp.float32)
    h1 = jnp.maximum(h1 + b1[...], 0.0)
    out[...] = hn[...] + jnp.dot(h1, w2[...],
                                 preferred_element_type=jnp.float32) + b2[...]


@jax.jit
def _tc_node(hn, parts, nu_W1, nu_b1, nu_W2, nu_b2):
    grid = (N_NODES // BN,)
    nb = pl.BlockSpec((BN, D), lambda i: (i, 0))
    full = lambda a, b: pl.BlockSpec((a, b), lambda i: (0, 0))
    return pl.pallas_call(
        _node_body,
        grid=grid,
        in_specs=[
            nb,
            pl.BlockSpec((1, BN, D), lambda i: (0, i, 0)),
            pl.BlockSpec((1, BN, D), lambda i: (1, i, 0)),
            full(D, H1), full(D, H1), full(1, H1),
            full(H1, D), full(1, D),
        ],
        out_specs=nb,
        out_shape=jax.ShapeDtypeStruct((N_NODES, D), jnp.float32),
    )(hn, parts, parts, nu_W1[:D], nu_W1[D:], nu_b1.reshape(1, H1),
      nu_W2, nu_b2.reshape(1, D))


def kernel(hn, he, edge_index, fe, fes, norm, ev_W1, ev_b1, ev_W2, ev_b2,
           fc_W1, fc_W2, nu_W1, nu_b1, nu_W2, nu_b2):
    src = edge_index[0]
    dst = edge_index[1]
    hs, hd = _sc_gather(hn, src, dst)
    hen, hen_s = _tc_edge(he, hs, hd, fe, fes, norm,
                          ev_W1, ev_b1, ev_W2, ev_b2, fc_W1, fc_W2)
    parts = _sc_scatter(hen_s, dst)
    hnn = _tc_node(hn, parts, nu_W1, nu_b1, nu_W2, nu_b2)
    return (hnn, hen)


# speedup vs baseline: 1.6325x; 1.0054x over previous
import functools

import jax
import jax.numpy as jnp
from jax import lax
from jax.experimental import pallas as pl
from jax.experimental.pallas import tpu as pltpu
from jax.experimental.pallas import tpu_sc as plsc

N_NODES = 10000
E = 160000
D = 128
D_VAL = 16
NUM_FES = 16
H1 = 512
H_FC = 64

NC = 2
NS = 16
NW = NC * NS

CHUNK = 128
N_CHUNKS = E // CHUNK
GATHER_ITERS = -(-N_CHUNKS // NW)
SC_CHUNKS = (E // 2) // CHUNK
SCATTER_ITERS = -(-SC_CHUNKS // NS)
ZROWS = 624
ZTAIL_OFF = ZROWS * NS
ZTAIL = N_NODES - ZTAIL_OFF


def _gather_body(hn_hbm, src_hbm, dst_hbm, hs_hbm, hd_hbm,
                 idx_a, rows_a, idx_b, rows_b, sem_a, sem_b):
    wid = lax.axis_index("s") * NC + lax.axis_index("c")

    def step(t, _):
        chunk = t * NW + wid

        @pl.when(chunk < N_CHUNKS)
        def _():
            base = chunk * CHUNK
            pltpu.sync_copy(src_hbm.at[pl.ds(base, CHUNK)], idx_a)
            pltpu.sync_copy(dst_hbm.at[pl.ds(base, CHUNK)], idx_b)
            cp_a = pltpu.async_copy(hn_hbm.at[idx_a], rows_a, sem_a)
            cp_b = pltpu.async_copy(hn_hbm.at[idx_b], rows_b, sem_b)
            cp_a.wait()
            pltpu.sync_copy(rows_a, hs_hbm.at[pl.ds(base, CHUNK)])
            cp_b.wait()
            pltpu.sync_copy(rows_b, hd_hbm.at[pl.ds(base, CHUNK)])
        return None

    lax.fori_loop(0, GATHER_ITERS, step, None)


@jax.jit
def _sc_gather(hn, src, dst):
    mesh = plsc.VectorSubcoreMesh(core_axis_name="c", subcore_axis_name="s")
    return pl.kernel(
        _gather_body,
        out_type=(
            jax.ShapeDtypeStruct((E, D), jnp.float32),
            jax.ShapeDtypeStruct((E, D), jnp.float32),
        ),
        mesh=mesh,
        scratch_types=[
            pltpu.VMEM((CHUNK,), jnp.int32),
            pltpu.VMEM((CHUNK, D), jnp.float32),
            pltpu.VMEM((CHUNK,), jnp.int32),
            pltpu.VMEM((CHUNK, D), jnp.float32),
            pltpu.SemaphoreType.DMA,
            pltpu.SemaphoreType.DMA,
        ],
    )(hn, src, dst)


def _scatter_body(hen_s_hbm, dst_hbm, zero_hbm, out_hbm,
                  idx_v, rows_v, acc):
    cid = lax.axis_index("c")
    sid = lax.axis_index("s")

    r0 = sid * ZROWS
    pltpu.sync_copy(zero_hbm.at[pl.ds(r0, ZROWS)], acc.at[pl.ds(r0, ZROWS)])

    @pl.when(sid == 0)
    def _():
        pltpu.sync_copy(zero_hbm.at[pl.ds(ZTAIL_OFF, ZTAIL)],
                        acc.at[pl.ds(ZTAIL_OFF, ZTAIL)])
    plsc.subcore_barrier()

    def step(t, _):
        chunk = t * NS + sid

        @pl.when(chunk < SC_CHUNKS)
        def _():
            base = cid * (E // 2) + chunk * CHUNK
            pltpu.sync_copy(dst_hbm.at[pl.ds(base, CHUNK)], idx_v)
            pltpu.sync_copy(hen_s_hbm.at[pl.ds(base, CHUNK)], rows_v)
            pltpu.sync_copy(rows_v, acc.at[idx_v], add=True)
        return None

    lax.fori_loop(0, SCATTER_ITERS, step, None)
    plsc.subcore_barrier()
    pltpu.sync_copy(acc.at[pl.ds(r0, ZROWS)], out_hbm.at[cid, pl.ds(r0, ZROWS)])

    @pl.when(sid == 0)
    def _():
        pltpu.sync_copy(acc.at[pl.ds(ZTAIL_OFF, ZTAIL)],
                        out_hbm.at[cid, pl.ds(ZTAIL_OFF, ZTAIL)])


@jax.jit
def _sc_scatter(hen_s, dst):
    mesh = plsc.VectorSubcoreMesh(core_axis_name="c", subcore_axis_name="s")
    zero = jnp.zeros((N_NODES, D), jnp.float32)
    return pl.kernel(
        _scatter_body,
        out_type=jax.ShapeDtypeStruct((NC, N_NODES, D), jnp.float32),
        mesh=mesh,
        scratch_types=[
            pltpu.VMEM((CHUNK,), jnp.int32),
            pltpu.VMEM((CHUNK, D), jnp.float32),
            pltpu.VMEM_SHARED((N_NODES, D), jnp.float32),
        ],
    )(hen_s, dst, zero)


BE = 800


def _edge_body(he, hs, hd, fe, fes, norm,
               w1a, w1b, w1c, b1, w2, b2, fcw1, fcw2,
               hen_out, hen_s_out):
    bf = jnp.bfloat16
    h1 = jnp.dot(he[...].astype(bf), w1a[...], preferred_element_type=jnp.float32)
    h1 += jnp.dot(hs[...].astype(bf), w1b[...], preferred_element_type=jnp.float32)
    h1 += jnp.dot(hd[...].astype(bf), w1c[...], preferred_element_type=jnp.float32)
    h1 = jnp.maximum(h1 + b1[...], 0.0)
    v = jnp.dot(h1.astype(bf), w2[...], preferred_element_type=jnp.float32) + b2[...]
    h = jnp.maximum(jnp.dot(fes[...].astype(bf), fcw1[...],
                            preferred_element_type=jnp.float32) * 0.25, 0.0)
    g = jnp.dot(h.astype(bf), fcw2[...], preferred_element_type=jnp.float32)
    acc = v[:, 0:1] * g[:, 0:D]
    for j in range(1, D_VAL):
        acc += v[:, j:j + 1] * g[:, j * D:(j + 1) * D]
    heu = fe[...] * acc * (1.0 / 32.0)
    hen = he[...] + heu
    hen_out[...] = hen
    hen_s_out[...] = hen * norm[...]


@jax.jit
def _tc_edge(he, hs, hd, fe, fes, norm, ev_W1, ev_b1, ev_W2, ev_b2,
             fc_W1, fc_W2):
    grid = (E // BE,)
    eb = lambda w: pl.BlockSpec((BE, w), lambda i: (i, 0))
    full = lambda a, b: pl.BlockSpec((a, b), lambda i: (0, 0))
    return pl.pallas_call(
        _edge_body,
        grid=grid,
        in_specs=[
            eb(D), eb(D), eb(D), eb(1), eb(NUM_FES), eb(1),
            full(D, H1), full(D, H1), full(D, H1), full(1, H1),
            full(H1, D_VAL), full(1, D_VAL),
            full(NUM_FES, H_FC), full(H_FC, D_VAL * D),
        ],
        out_specs=[eb(D), eb(D)],
        out_shape=(
            jax.ShapeDtypeStruct((E, D), jnp.float32),
            jax.ShapeDtypeStruct((E, D), jnp.float32),
        ),
    )(he, hs, hd, fe, fes, norm.reshape(E, 1),
      ev_W1[:D].astype(jnp.bfloat16), ev_W1[D:2 * D].astype(jnp.bfloat16),
      ev_W1[2 * D:].astype(jnp.bfloat16), ev_b1.reshape(1, H1),
      ev_W2.astype(jnp.bfloat16), ev_b2.reshape(1, D_VAL),
      fc_W1.astype(jnp.bfloat16), fc_W2.astype(jnp.bfloat16))


BN = 2000


def _node_body(hn, p0, p1, w1a, w1b, b1, w2, b2, out):
    bf = jnp.bfloat16
    ntmp = p0[0] + p1[0]
    h1 = jnp.dot(hn[...].astype(bf), w1a[...], preferred_element_type=jnp.float32)
    h1 += jnp.dot(ntmp.astype(bf), w1b[...], preferred_element_type=jnp.float32)
    h1 = jnp.maximum(h1 + b1[...], 0.0)
    out[...] = hn[...] + jnp.dot(h1.astype(bf), w2[...],
                                 preferred_element_type=jnp.float32) + b2[...]


@jax.jit
def _tc_node(hn, parts, nu_W1, nu_b1, nu_W2, nu_b2):
    grid = (N_NODES // BN,)
    nb = pl.BlockSpec((BN, D), lambda i: (i, 0))
    full = lambda a, b: pl.BlockSpec((a, b), lambda i: (0, 0))
    return pl.pallas_call(
        _node_body,
        grid=grid,
        in_specs=[
            nb,
            pl.BlockSpec((1, BN, D), lambda i: (0, i, 0)),
            pl.BlockSpec((1, BN, D), lambda i: (1, i, 0)),
            full(D, H1), full(D, H1), full(1, H1),
            full(H1, D), full(1, D),
        ],
        out_specs=nb,
        out_shape=jax.ShapeDtypeStruct((N_NODES, D), jnp.float32),
    )(hn, parts, parts, nu_W1[:D].astype(jnp.bfloat16),
      nu_W1[D:].astype(jnp.bfloat16), nu_b1.reshape(1, H1),
      nu_W2.astype(jnp.bfloat16), nu_b2.reshape(1, D))


def kernel(hn, he, edge_index, fe, fes, norm, ev_W1, ev_b1, ev_W2, ev_b2,
           fc_W1, fc_W2, nu_W1, nu_b1, nu_W2, nu_b2):
    src = edge_index[0]
    dst = edge_index[1]
    hs, hd = _sc_gather(hn, src, dst)
    hen, hen_s = _tc_edge(he, hs, hd, fe, fes, norm,
                          ev_W1, ev_b1, ev_W2, ev_b2, fc_W1, fc_W2)
    parts = _sc_scatter(hen_s, dst)
    hnn = _tc_node(hn, parts, nu_W1, nu_b1, nu_W2, nu_b2)
    return (hnn, hen)
